# baseline (speedup 1.0000x reference)
import functools

import jax
import jax.numpy as jnp
from jax import lax
from jax.experimental import pallas as pl
from jax.experimental.pallas import tpu as pltpu
from jax.experimental.pallas import tpu_sc as plsc

N = 10000
D = 128
E = 320000

NC = 2
NS = 16
NW = NC * NS
K = 128
NCH = 80
EW = NCH * K
EPAD = NW * EW
NT = 10240
RPW = NT // NS
RB = 1024
RBF = 1000

_mesh = plsc.VectorSubcoreMesh(core_axis_name="c", subcore_axis_name="s")
_sc_params = pltpu.CompilerParams(needs_layout_passes=False)


def _deg_body(dstc_hbm, out_hbm, idx_v, deg_v):
    c = lax.axis_index("c")
    s = lax.axis_index("s")
    w = s * NC + c
    pltpu.sync_copy(dstc_hbm.at[w], idx_v)
    zero16 = jnp.zeros((16,), jnp.float32)
    def zbody(i, _):
        deg_v[pl.ds(i * 16, 16)] = zero16
        return ()
    lax.fori_loop(0, NT // 16, zbody, ())
    one16 = jnp.ones((16,), jnp.float32)
    def body(i, _):
        plsc.addupdate_scatter(deg_v, [idx_v[i]], one16)
        return ()
    lax.fori_loop(0, EW // 16, body, ())
    pltpu.sync_copy(deg_v, out_hbm.at[w])


_deg_call = functools.partial(
    pl.kernel,
    out_type=jax.ShapeDtypeStruct((NW, NT), jnp.float32),
    mesh=_mesh,
    compiler_params=_sc_params,
    scratch_types=[
        pltpu.VMEM((EW // 16, 16), jnp.int32),
        pltpu.VMEM((NT,), jnp.float32),
    ],
)(_deg_body)


IB = 40
NBLK = NCH // IB
KH = K // 2


def _layer_body(hw_hbm, src_hbm, dst_hbm, out_hbm,
                src_v, dst_v, buf0, buf1, ia0, ib0, ia1, ib1,
                acc, sem0, sem1):
    c = lax.axis_index("c")
    s = lax.axis_index("s")
    w = s * NC + c

    zero16 = jnp.zeros((16,), jnp.float32)
    def zrow(i, _):
        for t in range(8):
            buf0[i, pl.ds(t * 16, 16)] = zero16
        return ()
    lax.fori_loop(0, K, zrow, ())
    for t in range(RPW // K):
        pltpu.sync_copy(buf0, acc.at[pl.ds(s * RPW + t * K, K)])
    plsc.subcore_barrier()

    cNT = c * NT
    def gstart(j, buf, sem, ia, ib):
        for q in range(KH // 16):
            ia[pl.ds(q * 16, 16)] = src_v[j, pl.ds(q * 16, 16)] + cNT
            ib[pl.ds(q * 16, 16)] = src_v[j, pl.ds(KH + q * 16, 16)] + cNT
        pltpu.make_async_copy(hw_hbm.at[ia], buf.at[pl.ds(0, KH)], sem).start()
        pltpu.make_async_copy(hw_hbm.at[ib], buf.at[pl.ds(KH, KH)],
                              sem).start()

    def gwait(buf, sem):
        pltpu.make_async_copy(hw_hbm.at[ia0], buf.at[pl.ds(0, KH)], sem).wait()
        pltpu.make_async_copy(hw_hbm.at[ia0], buf.at[pl.ds(KH, KH)],
                              sem).wait()

    def blk(b, _):
        base = w * NCH + b * IB
        pltpu.sync_copy(src_hbm.at[pl.ds(base, IB)], src_v)
        pltpu.sync_copy(dst_hbm.at[pl.ds(base, IB)], dst_v)
        gstart(0, buf0, sem0, ia0, ib0)
        def body(i, _):
            j = 2 * i
            gstart(j + 1, buf1, sem1, ia1, ib1)
            gwait(buf0, sem0)
            pltpu.sync_copy(buf0, acc.at[dst_v.at[j]], add=True)
            @pl.when(i < IB // 2 - 1)
            def _():
                gstart(j + 2, buf0, sem0, ia0, ib0)
            gwait(buf1, sem1)
            pltpu.sync_copy(buf1, acc.at[dst_v.at[j + 1]], add=True)
            return ()
        lax.fori_loop(0, IB // 2, body, ())
        return ()
    lax.fori_loop(0, NBLK, blk, ())

    plsc.subcore_barrier()
    pltpu.sync_copy(acc.at[pl.ds(s * RPW, RPW)],
                    out_hbm.at[pl.ds(c * NT + s * RPW, RPW)])


_layer_call = functools.partial(
    pl.kernel,
    out_type=jax.ShapeDtypeStruct((NC * NT, D), jnp.float32),
    mesh=_mesh,
    compiler_params=_sc_params,
    scratch_types=[
        pltpu.VMEM((IB, K), jnp.int32),
        pltpu.VMEM((IB, K), jnp.int32),
        pltpu.VMEM((K, D), jnp.float32),
        pltpu.VMEM((K, D), jnp.float32),
        pltpu.VMEM((KH,), jnp.int32),
        pltpu.VMEM((KH,), jnp.int32),
        pltpu.VMEM((KH,), jnp.int32),
        pltpu.VMEM((KH,), jnp.int32),
        pltpu.VMEM_SHARED((NT, D), jnp.float32),
        pltpu.SemaphoreType.DMA,
        pltpu.SemaphoreType.DMA,
    ],
)(_layer_body)


def _h0_kernel(x_ref, w_ref, b_ref, o_ref):
    h = jnp.dot(x_ref[...], w_ref[...], preferred_element_type=jnp.float32)
    o_ref[...] = jnp.maximum(h + b_ref[...], 0.0)


def _dis_kernel(degp_ref, o_ref):
    ones = jnp.ones((NW, 1), jnp.float32)
    s = lax.dot_general(degp_ref[...], ones, (((0,), (0,)), ((), ())),
                        preferred_element_type=jnp.float32)
    o_ref[...] = lax.rsqrt(s + 1.0)


def _hw1_kernel(h_ref, w_ref, dis_ref, o_ref):
    hw = jnp.dot(h_ref[...], w_ref[...], preferred_element_type=jnp.float32)
    o_ref[...] = hw * dis_ref[...]


def _comb_kernel(a0_ref, a1_ref, hwp_ref, dis_ref, b_ref, w_ref, o_ref):
    dis = dis_ref[...]
    pre = (a0_ref[...] + a1_ref[...] + hwp_ref[...]) * dis + b_ref[...]
    h = jnp.maximum(pre, 0.0)
    hw = jnp.dot(h, w_ref[...], preferred_element_type=jnp.float32)
    o_ref[...] = hw * dis


def _final_kernel(a0t_ref, a1t_ref, hwt_ref, dst_ref,
                  a0b_ref, a1b_ref, hwb_ref, dsb_ref,
                  b3_ref, w1a_ref, w1b_ref, bc1_ref, w2_ref, bc2_ref, o_ref):
    ht = jnp.maximum((a0t_ref[...] + a1t_ref[...] + hwt_ref[...])
                     * dst_ref[...] + b3_ref[...], 0.0)
    hb = jnp.maximum((a0b_ref[...] + a1b_ref[...] + hwb_ref[...])
                     * dsb_ref[...] + b3_ref[...], 0.0)
    z = jnp.dot(ht, w1a_ref[...], preferred_element_type=jnp.float32)
    z += jnp.dot(hb, w1b_ref[...], preferred_element_type=jnp.float32)
    z = jnp.maximum(z + bc1_ref[...], 0.0)
    o_ref[...] = jnp.dot(z, w2_ref[...],
                         preferred_element_type=jnp.float32) + bc2_ref[...]


def _row_spec(bs=RB):
    return pl.BlockSpec((bs, D), lambda i: (i, 0))


_full = lambda shp: pl.BlockSpec(shp, lambda i: (0, 0))


def _mm_h0(x, w, b):
    return pl.pallas_call(
        _h0_kernel,
        grid=(NT // RB,),
        in_specs=[_row_spec(), _full((D, D)), _full((1, D))],
        out_specs=_row_spec(),
        out_shape=jax.ShapeDtypeStruct((NT, D), jnp.float32),
    )(x, w, b)


def _mk_dis(degp):
    return pl.pallas_call(
        _dis_kernel,
        grid=(1,),
        in_specs=[_full((NW, NT))],
        out_specs=pl.BlockSpec((NT, 1), lambda i: (0, 0)),
        out_shape=jax.ShapeDtypeStruct((NT, 1), jnp.float32),
    )(degp)


def _dis_spec(bs=RB, off=0):
    return pl.BlockSpec((bs, 1), lambda i, off=off: (i + off, 0))


def _wrap():
    nb = NT // RB
    return pl.BlockSpec(
        (RB, D), lambda i: (jnp.where(i < nb, i, i - nb), 0))


def _wrap_dis():
    nb = NT // RB
    return pl.BlockSpec(
        (RB, 1), lambda i: (jnp.where(i < nb, i, i - nb), 0))


def _mm_hw1(h, w, dis):
    return pl.pallas_call(
        _hw1_kernel,
        grid=(2 * NT // RB,),
        in_specs=[_wrap(), _full((D, D)), _wrap_dis()],
        out_specs=_row_spec(),
        out_shape=jax.ShapeDtypeStruct((2 * NT, D), jnp.float32),
    )(h, w, dis)


def _mm_comb(acc0, acc1, hwp, dis, b, w):
    return pl.pallas_call(
        _comb_kernel,
        grid=(2 * NT // RB,),
        in_specs=[_wrap(), _wrap(), _wrap(), _wrap_dis(),
                  _full((1, D)), _full((D, D))],
        out_specs=_row_spec(),
        out_shape=jax.ShapeDtypeStruct((2 * NT, D), jnp.float32),
    )(acc0, acc1, hwp, dis, b, w)


def _mm_final(acc0, acc1, hwp, dis, b3, w1a, w1b, bc1, w2p, bc2p):
    bs = N // 2
    nb = bs // RBF
    top = pl.BlockSpec((RBF, D), lambda i: (i, 0))
    bot = pl.BlockSpec((RBF, D), lambda i: (i + nb, 0))
    return pl.pallas_call(
        _final_kernel,
        grid=(nb,),
        in_specs=[
            top, top, top, _dis_spec(RBF),
            bot, bot, bot, _dis_spec(RBF, nb),
            _full((1, D)), _full((D, D)), _full((D, D)),
            _full((1, D)), _full((D, D)), _full((1, D)),
        ],
        out_specs=pl.BlockSpec((RBF, D), lambda i: (i, 0)),
        out_shape=jax.ShapeDtypeStruct((bs, D), jnp.float32),
    )(acc0, acc1, hwp, dis, acc0, acc1, hwp, dis,
      b3, w1a, w1b, bc1, w2p, bc2p)


def kernel(x, edge_index, W_in, b_in, W_g1, b_g1, W_g2, b_g2, W_g3, b_g3,
           W_c1, b_c1, W_c2, b_c2):
    src = edge_index[0]
    dst = edge_index[1]
    pad = EPAD - E
    srcp = jnp.concatenate([src, jnp.zeros((pad,), jnp.int32)])
    dstp = jnp.concatenate([dst, jnp.full((pad,), N, jnp.int32)])
    src3 = srcp.reshape(NW * NCH, K)
    dst3 = dstp.reshape(NW * NCH, K)
    dstc = dstp.reshape(NW, EW // 16, 16)

    xp = jnp.pad(x, ((0, NT - N), (0, 0)))
    b_in2 = b_in.reshape(1, D)
    b_g12 = b_g1.reshape(1, D)
    b_g22 = b_g2.reshape(1, D)
    b_g32 = b_g3.reshape(1, D)
    bc12 = b_c1.reshape(1, D)
    w1a = W_c1[:D]
    w1b = W_c1[D:]
    w2p = jnp.pad(W_c2, ((0, 0), (0, D - W_c2.shape[1])))
    bc2p = jnp.pad(b_c2, (0, D - b_c2.shape[0])).reshape(1, D)

    def planes(a):
        r = a.reshape(NC, NT, D)
        return r[0], r[1]

    degp = _deg_call(dstc)
    h0 = _mm_h0(xp, W_in, b_in2)
    dis = _mk_dis(degp)
    hw1 = _mm_hw1(h0, W_g1, dis)
    a10, a11 = planes(_layer_call(hw1, src3, dst3))
    hw2 = _mm_comb(a10, a11, hw1, dis, b_g12, W_g2)
    a20, a21 = planes(_layer_call(hw2, src3, dst3))
    hw3 = _mm_comb(a20, a21, hw2, dis, b_g22, W_g3)
    a30, a31 = planes(_layer_call(hw3, src3, dst3))
    out = _mm_final(a30, a31, hw3, dis, b_g32, w1a, w1b, bc12, w2p, bc2p)
    return out[:, :2]


# speedup vs baseline: 1.0773x; 1.0773x over previous
import functools

import jax
import jax.numpy as jnp
from jax import lax
from jax.experimental import pallas as pl
from jax.experimental.pallas import tpu as pltpu
from jax.experimental.pallas import tpu_sc as plsc

N = 10000
D = 128
E = 320000

NC = 2
NS = 16
NW = NC * NS
K = 128
NCH = 80
EW = NCH * K
EPAD = NW * EW
NT = 10240
RPW = NT // NS
RB = 1024
RBF = 1000

_mesh = plsc.VectorSubcoreMesh(core_axis_name="c", subcore_axis_name="s")
_sc_params = pltpu.CompilerParams(needs_layout_passes=False)


def _deg_body(dstc_hbm, out_hbm, idx_v, deg_v):
    c = lax.axis_index("c")
    s = lax.axis_index("s")
    w = s * NC + c
    pltpu.sync_copy(dstc_hbm.at[w], idx_v)
    zero16 = jnp.zeros((16,), jnp.float32)
    def zbody(i, _):
        deg_v[pl.ds(i * 16, 16)] = zero16
        return ()
    lax.fori_loop(0, NT // 16, zbody, ())
    one16 = jnp.ones((16,), jnp.float32)
    def body(i, _):
        plsc.addupdate_scatter(deg_v, [idx_v[i]], one16)
        return ()
    lax.fori_loop(0, EW // 16, body, ())
    pltpu.sync_copy(deg_v, out_hbm.at[w])


_deg_call = functools.partial(
    pl.kernel,
    out_type=jax.ShapeDtypeStruct((NW, NT), jnp.float32),
    mesh=_mesh,
    compiler_params=_sc_params,
    scratch_types=[
        pltpu.VMEM((EW // 16, 16), jnp.int32),
        pltpu.VMEM((NT,), jnp.float32),
    ],
)(_deg_body)


IB = 40
NBLK = NCH // IB
KH = K // 2


def _layer_body(hw_hbm, src_hbm, dst_hbm, out_hbm,
                src_v, dst_v, buf0, buf1, ia0, ib0, ia1, ib1,
                acc, sem0, sem1):
    c = lax.axis_index("c")
    s = lax.axis_index("s")
    w = s * NC + c

    zero16 = jnp.zeros((16,), jnp.float32)
    def zrow(i, _):
        for t in range(8):
            buf0[i, pl.ds(t * 16, 16)] = zero16
        return ()
    lax.fori_loop(0, K, zrow, ())
    for t in range(RPW // K):
        pltpu.sync_copy(buf0, acc.at[pl.ds(s * RPW + t * K, K)])
    plsc.subcore_barrier()

    def gstart(j, buf, sem, ia, ib):
        for q in range(KH // 16):
            ia[pl.ds(q * 16, 16)] = src_v[j, pl.ds(q * 16, 16)]
            ib[pl.ds(q * 16, 16)] = src_v[j, pl.ds(KH + q * 16, 16)]
        pltpu.make_async_copy(hw_hbm.at[ia], buf.at[pl.ds(0, KH)], sem).start()
        pltpu.make_async_copy(hw_hbm.at[ib], buf.at[pl.ds(KH, KH)],
                              sem).start()

    def gwait(buf, sem):
        pltpu.make_async_copy(hw_hbm.at[ia0], buf.at[pl.ds(0, KH)], sem).wait()
        pltpu.make_async_copy(hw_hbm.at[ia0], buf.at[pl.ds(KH, KH)],
                              sem).wait()

    def blk(b, _):
        base = w * NCH + b * IB
        pltpu.sync_copy(src_hbm.at[pl.ds(base, IB)], src_v)
        pltpu.sync_copy(dst_hbm.at[pl.ds(base, IB)], dst_v)
        gstart(0, buf0, sem0, ia0, ib0)
        def body(i, _):
            j = 2 * i
            gstart(j + 1, buf1, sem1, ia1, ib1)
            gwait(buf0, sem0)
            pltpu.sync_copy(buf0, acc.at[dst_v.at[j]], add=True)
            @pl.when(i < IB // 2 - 1)
            def _():
                gstart(j + 2, buf0, sem0, ia0, ib0)
            gwait(buf1, sem1)
            pltpu.sync_copy(buf1, acc.at[dst_v.at[j + 1]], add=True)
            return ()
        lax.fori_loop(0, IB // 2, body, ())
        return ()
    lax.fori_loop(0, NBLK, blk, ())

    plsc.subcore_barrier()
    def ostart(t, buf, sem):
        pltpu.make_async_copy(acc.at[pl.ds(s * RPW + t * K, K)], buf,
                              sem).start()

    def owait(buf, sem):
        pltpu.make_async_copy(acc.at[pl.ds(s * RPW, K)], buf, sem).wait()

    ostart(0, buf0, sem0)
    for t in range(RPW // K):
        buf, sem = (buf0, sem0) if t % 2 == 0 else (buf1, sem1)
        if t + 1 < RPW // K:
            nbuf, nsem = (buf1, sem1) if t % 2 == 0 else (buf0, sem0)
            ostart(t + 1, nbuf, nsem)
        owait(buf, sem)
        pltpu.sync_copy(buf, out_hbm.at[pl.ds(c * NT + s * RPW + t * K, K)])


_layer_call = functools.partial(
    pl.kernel,
    out_type=jax.ShapeDtypeStruct((NC * NT, D), jnp.float32),
    mesh=_mesh,
    compiler_params=_sc_params,
    scratch_types=[
        pltpu.VMEM((IB, K), jnp.int32),
        pltpu.VMEM((IB, K), jnp.int32),
        pltpu.VMEM((K, D), jnp.float32),
        pltpu.VMEM((K, D), jnp.float32),
        pltpu.VMEM((KH,), jnp.int32),
        pltpu.VMEM((KH,), jnp.int32),
        pltpu.VMEM((KH,), jnp.int32),
        pltpu.VMEM((KH,), jnp.int32),
        pltpu.VMEM_SHARED((NT, D), jnp.float32),
        pltpu.SemaphoreType.DMA,
        pltpu.SemaphoreType.DMA,
    ],
)(_layer_body)


def _h0_kernel(x_ref, w_ref, b_ref, o_ref):
    h = jnp.dot(x_ref[...], w_ref[...], preferred_element_type=jnp.float32)
    o_ref[...] = jnp.maximum(h + b_ref[...], 0.0)


def _dis_kernel(degp_ref, o_ref):
    ones = jnp.ones((NW, 1), jnp.float32)
    s = lax.dot_general(degp_ref[...], ones, (((0,), (0,)), ((), ())),
                        preferred_element_type=jnp.float32)
    o_ref[...] = lax.rsqrt(s + 1.0)


def _hw1_kernel(h_ref, w_ref, dis_ref, o_ref):
    hw = jnp.dot(h_ref[...], w_ref[...], preferred_element_type=jnp.float32)
    o_ref[...] = hw * dis_ref[...]


def _comb_kernel(a0_ref, a1_ref, hwp_ref, dis_ref, b_ref, w_ref, o_ref):
    dis = dis_ref[...]
    pre = (a0_ref[...] + a1_ref[...] + hwp_ref[...]) * dis + b_ref[...]
    h = jnp.maximum(pre, 0.0)
    hw = jnp.dot(h, w_ref[...], preferred_element_type=jnp.float32)
    o_ref[...] = hw * dis


def _final_kernel(a0t_ref, a1t_ref, hwt_ref, dst_ref,
                  a0b_ref, a1b_ref, hwb_ref, dsb_ref,
                  b3_ref, w1a_ref, w1b_ref, bc1_ref, w2_ref, bc2_ref, o_ref):
    ht = jnp.maximum((a0t_ref[...] + a1t_ref[...] + hwt_ref[...])
                     * dst_ref[...] + b3_ref[...], 0.0)
    hb = jnp.maximum((a0b_ref[...] + a1b_ref[...] + hwb_ref[...])
                     * dsb_ref[...] + b3_ref[...], 0.0)
    z = jnp.dot(ht, w1a_ref[...], preferred_element_type=jnp.float32)
    z += jnp.dot(hb, w1b_ref[...], preferred_element_type=jnp.float32)
    z = jnp.maximum(z + bc1_ref[...], 0.0)
    o_ref[...] = jnp.dot(z, w2_ref[...],
                         preferred_element_type=jnp.float32) + bc2_ref[...]


def _row_spec(bs=RB):
    return pl.BlockSpec((bs, D), lambda i: (i, 0))


_full = lambda shp: pl.BlockSpec(shp, lambda i: (0, 0))


def _mm_h0(x, w, b):
    return pl.pallas_call(
        _h0_kernel,
        grid=(NT // RB,),
        in_specs=[_row_spec(), _full((D, D)), _full((1, D))],
        out_specs=_row_spec(),
        out_shape=jax.ShapeDtypeStruct((NT, D), jnp.float32),
    )(x, w, b)


def _mk_dis(degp):
    return pl.pallas_call(
        _dis_kernel,
        grid=(1,),
        in_specs=[_full((NW, NT))],
        out_specs=pl.BlockSpec((NT, 1), lambda i: (0, 0)),
        out_shape=jax.ShapeDtypeStruct((NT, 1), jnp.float32),
    )(degp)


def _dis_spec(bs=RB, off=0):
    return pl.BlockSpec((bs, 1), lambda i, off=off: (i + off, 0))


def _mm_hw1(h, w, dis):
    return pl.pallas_call(
        _hw1_kernel,
        grid=(NT // RB,),
        in_specs=[_row_spec(), _full((D, D)), _dis_spec()],
        out_specs=_row_spec(),
        out_shape=jax.ShapeDtypeStruct((NT, D), jnp.float32),
    )(h, w, dis)


def _mm_comb(acc0, acc1, hwp, dis, b, w):
    return pl.pallas_call(
        _comb_kernel,
        grid=(NT // RB,),
        in_specs=[_row_spec(), _row_spec(), _row_spec(), _dis_spec(),
                  _full((1, D)), _full((D, D))],
        out_specs=_row_spec(),
        out_shape=jax.ShapeDtypeStruct((NT, D), jnp.float32),
    )(acc0, acc1, hwp, dis, b, w)


def _mm_final(acc0, acc1, hwp, dis, b3, w1a, w1b, bc1, w2p, bc2p):
    bs = N // 2
    nb = bs // RBF
    top = pl.BlockSpec((RBF, D), lambda i: (i, 0))
    bot = pl.BlockSpec((RBF, D), lambda i: (i + nb, 0))
    return pl.pallas_call(
        _final_kernel,
        grid=(nb,),
        in_specs=[
            top, top, top, _dis_spec(RBF),
            bot, bot, bot, _dis_spec(RBF, nb),
            _full((1, D)), _full((D, D)), _full((D, D)),
            _full((1, D)), _full((D, D)), _full((1, D)),
        ],
        out_specs=pl.BlockSpec((RBF, D), lambda i: (i, 0)),
        out_shape=jax.ShapeDtypeStruct((bs, D), jnp.float32),
    )(acc0, acc1, hwp, dis, acc0, acc1, hwp, dis,
      b3, w1a, w1b, bc1, w2p, bc2p)


def kernel(x, edge_index, W_in, b_in, W_g1, b_g1, W_g2, b_g2, W_g3, b_g3,
           W_c1, b_c1, W_c2, b_c2):
    src = edge_index[0]
    dst = edge_index[1]
    pad = EPAD - E
    srcp = jnp.concatenate([src, jnp.zeros((pad,), jnp.int32)])
    dstp = jnp.concatenate([dst, jnp.full((pad,), N, jnp.int32)])
    src3 = srcp.reshape(NW * NCH, K)
    dst3 = dstp.reshape(NW * NCH, K)
    dstc = dstp.reshape(NW, EW // 16, 16)

    xp = jnp.pad(x, ((0, NT - N), (0, 0)))
    b_in2 = b_in.reshape(1, D)
    b_g12 = b_g1.reshape(1, D)
    b_g22 = b_g2.reshape(1, D)
    b_g32 = b_g3.reshape(1, D)
    bc12 = b_c1.reshape(1, D)
    w1a = W_c1[:D]
    w1b = W_c1[D:]
    w2p = jnp.pad(W_c2, ((0, 0), (0, D - W_c2.shape[1])))
    bc2p = jnp.pad(b_c2, (0, D - b_c2.shape[0])).reshape(1, D)

    def planes(a):
        r = a.reshape(NC, NT, D)
        return r[0], r[1]

    degp = _deg_call(dstc)
    h0 = _mm_h0(xp, W_in, b_in2)
    dis = _mk_dis(degp)
    hw1 = _mm_hw1(h0, W_g1, dis)
    a10, a11 = planes(_layer_call(hw1, src3, dst3))
    hw2 = _mm_comb(a10, a11, hw1, dis, b_g12, W_g2)
    a20, a21 = planes(_layer_call(hw2, src3, dst3))
    hw3 = _mm_comb(a20, a21, hw2, dis, b_g22, W_g3)
    a30, a31 = planes(_layer_call(hw3, src3, dst3))
    out = _mm_final(a30, a31, hw3, dis, b_g32, w1a, w1b, bc12, w2p, bc2p)
    return out[:, :2]
